# baseline (speedup 1.0000x reference)
import functools

import jax
import jax.numpy as jnp
from jax import lax
from jax.experimental import pallas as pl
from jax.experimental.pallas import tpu as pltpu
from jax.experimental.pallas import tpu_sc as plsc

_NB = 31
_L = 16
_NW = 32
_CHUNK = 32768
_UNROLL = 4

_mesh = plsc.VectorSubcoreMesh(core_axis_name="c", subcore_axis_name="s")


def _wid():
    return lax.axis_index("c") * 16 + lax.axis_index("s")


def _stats_body(x_hbm, out_hbm, buf0, buf1, stats_v, sem0, sem1, *, npw, nchunks):
    base = _wid() * npw
    bufs = (buf0, buf1)
    sems = (sem0, sem1)

    def copy(c):
        return pltpu.make_async_copy(
            x_hbm.at[pl.ds(base + c * _CHUNK, _CHUNK)], bufs[c % 2], sems[c % 2])

    copy(0).start()
    mn = jnp.full((_L,), jnp.inf, jnp.float32)
    mx = jnp.full((_L,), -jnp.inf, jnp.float32)
    s = jnp.zeros((_L,), jnp.float32)
    ss = jnp.zeros((_L,), jnp.float32)
    carry = (mn, mx, s, ss)

    for c in range(nchunks):
        if c + 1 < nchunks:
            copy(c + 1).start()
        copy(c).wait()
        buf = bufs[c % 2]

        def body(i, carry, buf=buf):
            mn, mx, s, ss = carry
            for k in range(_UNROLL):
                v = buf[pl.ds((i * _UNROLL + k) * _L, _L)]
                mn = jnp.minimum(mn, v)
                mx = jnp.maximum(mx, v)
                s = s + v
                ss = ss + v * v
            return (mn, mx, s, ss)

        carry = lax.fori_loop(0, _CHUNK // (_UNROLL * _L), body, carry)

    mn, mx, s, ss = carry
    stats_v[pl.ds(0, _L)] = mn
    stats_v[pl.ds(_L, _L)] = mx
    stats_v[pl.ds(2 * _L, _L)] = s
    stats_v[pl.ds(3 * _L, _L)] = ss
    pltpu.sync_copy(stats_v, out_hbm.at[pl.ds(_wid() * 4 * _L, 4 * _L)])


def _hist_body(x_hbm, params_hbm, out_hbm, buf0, buf1, params_v, hist_v,
               sem0, sem1, *, npw, nchunks):
    base = _wid() * npw
    bufs = (buf0, buf1)
    sems = (sem0, sem1)

    def copy(c):
        return pltpu.make_async_copy(
            x_hbm.at[pl.ds(base + c * _CHUNK, _CHUNK)], bufs[c % 2], sems[c % 2])

    copy(0).start()
    pltpu.sync_copy(params_hbm, params_v)
    r0 = params_v[pl.ds(0, _L)]
    scl = params_v[pl.ds(_L, _L)]
    lane = lax.iota(jnp.int32, _L)
    ones = jnp.full((_L,), 1.0, jnp.float32)
    zero = jnp.zeros((_L,), jnp.float32)

    def zbody(i, _):
        hist_v[pl.ds(i * _L, _L)] = zero
        return 0

    lax.fori_loop(0, 32, zbody, 0)

    for c in range(nchunks):
        if c + 1 < nchunks:
            copy(c + 1).start()
        copy(c).wait()
        buf = bufs[c % 2]

        def body(i, _, buf=buf):
            for k in range(_UNROLL):
                v = buf[pl.ds((i * _UNROLL + k) * _L, _L)]
                t = (v - r0) * scl
                ix = t.astype(jnp.int32)
                ix = jnp.minimum(jnp.maximum(ix, 0), _NB - 1)
                addr = ix * _L + lane
                plsc.addupdate_scatter(hist_v, [addr], ones)
            return 0

        lax.fori_loop(0, _CHUNK // (_UNROLL * _L), body, 0)

    pltpu.sync_copy(hist_v, out_hbm.at[pl.ds(_wid() * 32 * _L, 32 * _L)])


def kernel(array):
    n = array.size
    npw = n // _NW
    nchunks = npw // _CHUNK

    stats_call = pl.kernel(
        functools.partial(_stats_body, npw=npw, nchunks=nchunks),
        out_type=jax.ShapeDtypeStruct((_NW * 4 * _L,), jnp.float32),
        mesh=_mesh,
        compiler_params=pltpu.CompilerParams(needs_layout_passes=False),
        scratch_types=[
            pltpu.VMEM((_CHUNK,), jnp.float32),
            pltpu.VMEM((_CHUNK,), jnp.float32),
            pltpu.VMEM((4 * _L,), jnp.float32),
            pltpu.SemaphoreType.DMA,
            pltpu.SemaphoreType.DMA,
        ],
    )
    stats = stats_call(array).reshape(_NW, 4, _L)
    mn = stats[:, 0, :].min()
    mx = stats[:, 1, :].max()
    s = stats[:, 2, :].sum()
    ss = stats[:, 3, :].sum()
    num = jnp.asarray(n, jnp.int32)

    r0 = jnp.where(mx == mn, mn - 0.5, mn)
    r1 = jnp.where(mx == mn, mx + 0.5, mx)
    edges = jnp.linspace(r0, r1, _NB + 1, dtype=jnp.float32)
    scale = jnp.float32(_NB) / (r1 - r0)
    params = jnp.concatenate([
        jnp.full((_L,), r0, jnp.float32),
        jnp.full((_L,), scale, jnp.float32),
    ])

    hist_call = pl.kernel(
        functools.partial(_hist_body, npw=npw, nchunks=nchunks),
        out_type=jax.ShapeDtypeStruct((_NW * 32 * _L,), jnp.float32),
        mesh=_mesh,
        compiler_params=pltpu.CompilerParams(needs_layout_passes=False),
        scratch_types=[
            pltpu.VMEM((_CHUNK,), jnp.float32),
            pltpu.VMEM((_CHUNK,), jnp.float32),
            pltpu.VMEM((2 * _L,), jnp.float32),
            pltpu.VMEM((32 * _L,), jnp.float32),
            pltpu.SemaphoreType.DMA,
            pltpu.SemaphoreType.DMA,
        ],
    )
    tables = hist_call(array, params).reshape(_NW, 32, _L)
    counts = tables.sum(axis=(0, 2))[:_NB]
    return (mn, mx, num, s, ss, edges, counts)


# speedup vs baseline: 1.1455x; 1.1455x over previous
import functools

import jax
import jax.numpy as jnp
from jax import lax
from jax.experimental import pallas as pl
from jax.experimental.pallas import tpu as pltpu
from jax.experimental.pallas import tpu_sc as plsc

_NB = 31
_L = 16
_NW = 32
_CHUNK = 32768
_UNROLL = 4

_mesh = plsc.VectorSubcoreMesh(core_axis_name="c", subcore_axis_name="s")


def _wid():
    return lax.axis_index("c") * 16 + lax.axis_index("s")


def _stats_body(x_hbm, out_hbm, buf0, buf1, stats_v, sem0, sem1, *, npw, nchunks):
    base = _wid() * npw
    bufs = (buf0, buf1)
    sems = (sem0, sem1)

    def copy(c):
        return pltpu.make_async_copy(
            x_hbm.at[pl.ds(base + c * _CHUNK, _CHUNK)], bufs[c % 2], sems[c % 2])

    copy(0).start()
    mn = jnp.full((_L,), jnp.inf, jnp.float32)
    mx = jnp.full((_L,), -jnp.inf, jnp.float32)
    s = jnp.zeros((_L,), jnp.float32)
    ss = jnp.zeros((_L,), jnp.float32)
    carry = (mn, mx, s, ss)

    for c in range(nchunks):
        if c + 1 < nchunks:
            copy(c + 1).start()
        copy(c).wait()
        buf = bufs[c % 2]

        def body(i, carry, buf=buf):
            mn, mx, s, ss = carry
            for k in range(_UNROLL):
                v = buf[pl.ds((i * _UNROLL + k) * _L, _L)]
                mn = jnp.minimum(mn, v)
                mx = jnp.maximum(mx, v)
                s = s + v
                ss = ss + v * v
            return (mn, mx, s, ss)

        carry = lax.fori_loop(0, _CHUNK // (_UNROLL * _L), body, carry)

    mn, mx, s, ss = carry
    stats_v[pl.ds(0, _L)] = mn
    stats_v[pl.ds(_L, _L)] = mx
    stats_v[pl.ds(2 * _L, _L)] = s
    stats_v[pl.ds(3 * _L, _L)] = ss
    pltpu.sync_copy(stats_v, out_hbm.at[pl.ds(_wid() * 4 * _L, 4 * _L)])


def _hist_body(x_hbm, params_hbm, out_hbm, buf0, buf1, params_v, hist_v,
               sem0, sem1, *, npw, nchunks):
    base = _wid() * npw
    bufs = (buf0, buf1)
    sems = (sem0, sem1)

    def copy(c):
        return pltpu.make_async_copy(
            x_hbm.at[pl.ds(base + c * _CHUNK, _CHUNK)], bufs[c % 2], sems[c % 2])

    copy(0).start()
    pltpu.sync_copy(params_hbm, params_v)
    r0 = params_v[pl.ds(0, _L)]
    scl = params_v[pl.ds(_L, _L)]
    lanes = [lax.iota(jnp.int32, _L) + k * 32 * _L for k in range(_UNROLL)]
    ones = jnp.full((_L,), 1.0, jnp.float32)
    zero = jnp.zeros((_L,), jnp.float32)

    def zbody(i, _):
        hist_v[pl.ds(i * _L, _L)] = zero
        return 0

    lax.fori_loop(0, _UNROLL * 32, zbody, 0)

    for c in range(nchunks):
        if c + 1 < nchunks:
            copy(c + 1).start()
        copy(c).wait()
        buf = bufs[c % 2]

        def body(i, _, buf=buf):
            for k in range(_UNROLL):
                v = buf[pl.ds((i * _UNROLL + k) * _L, _L)]
                t = (v - r0) * scl
                ix = t.astype(jnp.int32)
                addr = ix * _L + lanes[k]
                plsc.addupdate_scatter(hist_v, [addr], ones)
            return 0

        lax.fori_loop(0, _CHUNK // (_UNROLL * _L), body, 0)

    pltpu.sync_copy(
        hist_v, out_hbm.at[pl.ds(_wid() * _UNROLL * 32 * _L, _UNROLL * 32 * _L)])


def kernel(array):
    n = array.size
    npw = n // _NW
    nchunks = npw // _CHUNK

    stats_call = pl.kernel(
        functools.partial(_stats_body, npw=npw, nchunks=nchunks),
        out_type=jax.ShapeDtypeStruct((_NW * 4 * _L,), jnp.float32),
        mesh=_mesh,
        compiler_params=pltpu.CompilerParams(needs_layout_passes=False),
        scratch_types=[
            pltpu.VMEM((_CHUNK,), jnp.float32),
            pltpu.VMEM((_CHUNK,), jnp.float32),
            pltpu.VMEM((4 * _L,), jnp.float32),
            pltpu.SemaphoreType.DMA,
            pltpu.SemaphoreType.DMA,
        ],
    )
    stats = stats_call(array).reshape(_NW, 4, _L)
    mn = stats[:, 0, :].min()
    mx = stats[:, 1, :].max()
    s = stats[:, 2, :].sum()
    ss = stats[:, 3, :].sum()
    num = jnp.asarray(n, jnp.int32)

    r0 = jnp.where(mx == mn, mn - 0.5, mn)
    r1 = jnp.where(mx == mn, mx + 0.5, mx)
    edges = jnp.linspace(r0, r1, _NB + 1, dtype=jnp.float32)
    scale = jnp.float32(_NB) / (r1 - r0)
    params = jnp.concatenate([
        jnp.full((_L,), r0, jnp.float32),
        jnp.full((_L,), scale, jnp.float32),
    ])

    hist_call = pl.kernel(
        functools.partial(_hist_body, npw=npw, nchunks=nchunks),
        out_type=jax.ShapeDtypeStruct((_NW * _UNROLL * 32 * _L,), jnp.float32),
        mesh=_mesh,
        compiler_params=pltpu.CompilerParams(needs_layout_passes=False),
        scratch_types=[
            pltpu.VMEM((_CHUNK,), jnp.float32),
            pltpu.VMEM((_CHUNK,), jnp.float32),
            pltpu.VMEM((2 * _L,), jnp.float32),
            pltpu.VMEM((_UNROLL * 32 * _L,), jnp.float32),
            pltpu.SemaphoreType.DMA,
            pltpu.SemaphoreType.DMA,
        ],
    )
    tables = hist_call(array, params).reshape(_NW * _UNROLL, 32, _L)
    rows = tables.sum(axis=(0, 2))
    counts = rows[:_NB].at[_NB - 1].add(rows[_NB])
    return (mn, mx, num, s, ss, edges, counts)


# speedup vs baseline: 1.1501x; 1.0040x over previous
import functools

import jax
import jax.numpy as jnp
from jax import lax
from jax.experimental import pallas as pl
from jax.experimental.pallas import tpu as pltpu
from jax.experimental.pallas import tpu_sc as plsc

_NB = 31
_L = 16
_NW = 32
_CHUNK = 32768
_UNROLL = 8

_mesh = plsc.VectorSubcoreMesh(core_axis_name="c", subcore_axis_name="s")


def _wid():
    return lax.axis_index("c") * 16 + lax.axis_index("s")


def _stats_body(x_hbm, out_hbm, buf0, buf1, stats_v, sem0, sem1, *, npw, nchunks):
    base = _wid() * npw
    bufs = (buf0, buf1)
    sems = (sem0, sem1)

    def copy(c):
        return pltpu.make_async_copy(
            x_hbm.at[pl.ds(base + c * _CHUNK, _CHUNK)], bufs[c % 2], sems[c % 2])

    copy(0).start()
    mn = jnp.full((_L,), jnp.inf, jnp.float32)
    mx = jnp.full((_L,), -jnp.inf, jnp.float32)
    s = jnp.zeros((_L,), jnp.float32)
    ss = jnp.zeros((_L,), jnp.float32)
    carry = (mn, mx, s, ss)

    for c in range(nchunks):
        if c + 1 < nchunks:
            copy(c + 1).start()
        copy(c).wait()
        buf = bufs[c % 2]

        def body(i, carry, buf=buf):
            mn, mx, s, ss = carry
            for k in range(_UNROLL):
                v = buf[pl.ds((i * _UNROLL + k) * _L, _L)]
                mn = jnp.minimum(mn, v)
                mx = jnp.maximum(mx, v)
                s = s + v
                ss = ss + v * v
            return (mn, mx, s, ss)

        carry = lax.fori_loop(0, _CHUNK // (_UNROLL * _L), body, carry)

    mn, mx, s, ss = carry
    stats_v[pl.ds(0, _L)] = mn
    stats_v[pl.ds(_L, _L)] = mx
    stats_v[pl.ds(2 * _L, _L)] = s
    stats_v[pl.ds(3 * _L, _L)] = ss
    pltpu.sync_copy(stats_v, out_hbm.at[pl.ds(_wid() * 4 * _L, 4 * _L)])


def _hist_body(x_hbm, params_hbm, out_hbm, buf0, buf1, params_v, hist_v,
               sem0, sem1, *, npw, nchunks):
    base = _wid() * npw
    bufs = (buf0, buf1)
    sems = (sem0, sem1)

    def copy(c):
        return pltpu.make_async_copy(
            x_hbm.at[pl.ds(base + c * _CHUNK, _CHUNK)], bufs[c % 2], sems[c % 2])

    copy(0).start()
    pltpu.sync_copy(params_hbm, params_v)
    r0 = params_v[pl.ds(0, _L)]
    scl = params_v[pl.ds(_L, _L)]
    lanes = [lax.iota(jnp.int32, _L) + k * 32 * _L for k in range(_UNROLL)]
    ones = jnp.full((_L,), 1.0, jnp.float32)
    zero = jnp.zeros((_L,), jnp.float32)

    def zbody(i, _):
        hist_v[pl.ds(i * _L, _L)] = zero
        return 0

    lax.fori_loop(0, _UNROLL * 32, zbody, 0)

    for c in range(nchunks):
        if c + 1 < nchunks:
            copy(c + 1).start()
        copy(c).wait()
        buf = bufs[c % 2]

        def body(i, _, buf=buf):
            for k in range(_UNROLL):
                v = buf[pl.ds((i * _UNROLL + k) * _L, _L)]
                t = (v - r0) * scl
                ix = t.astype(jnp.int32)
                addr = ix * _L + lanes[k]
                plsc.addupdate_scatter(hist_v, [addr], ones)
            return 0

        lax.fori_loop(0, _CHUNK // (_UNROLL * _L), body, 0)

    pltpu.sync_copy(
        hist_v, out_hbm.at[pl.ds(_wid() * _UNROLL * 32 * _L, _UNROLL * 32 * _L)])


def kernel(array):
    n = array.size
    npw = n // _NW
    nchunks = npw // _CHUNK

    stats_call = pl.kernel(
        functools.partial(_stats_body, npw=npw, nchunks=nchunks),
        out_type=jax.ShapeDtypeStruct((_NW * 4 * _L,), jnp.float32),
        mesh=_mesh,
        compiler_params=pltpu.CompilerParams(needs_layout_passes=False),
        scratch_types=[
            pltpu.VMEM((_CHUNK,), jnp.float32),
            pltpu.VMEM((_CHUNK,), jnp.float32),
            pltpu.VMEM((4 * _L,), jnp.float32),
            pltpu.SemaphoreType.DMA,
            pltpu.SemaphoreType.DMA,
        ],
    )
    stats = stats_call(array).reshape(_NW, 4, _L)
    mn = stats[:, 0, :].min()
    mx = stats[:, 1, :].max()
    s = stats[:, 2, :].sum()
    ss = stats[:, 3, :].sum()
    num = jnp.asarray(n, jnp.int32)

    r0 = jnp.where(mx == mn, mn - 0.5, mn)
    r1 = jnp.where(mx == mn, mx + 0.5, mx)
    edges = jnp.linspace(r0, r1, _NB + 1, dtype=jnp.float32)
    scale = jnp.float32(_NB) / (r1 - r0)
    params = jnp.concatenate([
        jnp.full((_L,), r0, jnp.float32),
        jnp.full((_L,), scale, jnp.float32),
    ])

    hist_call = pl.kernel(
        functools.partial(_hist_body, npw=npw, nchunks=nchunks),
        out_type=jax.ShapeDtypeStruct((_NW * _UNROLL * 32 * _L,), jnp.float32),
        mesh=_mesh,
        compiler_params=pltpu.CompilerParams(needs_layout_passes=False),
        scratch_types=[
            pltpu.VMEM((_CHUNK,), jnp.float32),
            pltpu.VMEM((_CHUNK,), jnp.float32),
            pltpu.VMEM((2 * _L,), jnp.float32),
            pltpu.VMEM((_UNROLL * 32 * _L,), jnp.float32),
            pltpu.SemaphoreType.DMA,
            pltpu.SemaphoreType.DMA,
        ],
    )
    tables = hist_call(array, params).reshape(_NW * _UNROLL, 32, _L)
    rows = tables.sum(axis=(0, 2))
    counts = rows[:_NB].at[_NB - 1].add(rows[_NB])
    return (mn, mx, num, s, ss, edges, counts)
